# baseline (speedup 1.0000x reference)
import functools

import jax
import jax.numpy as jnp
from jax import lax
from jax.experimental import pallas as pl
from jax.experimental.pallas import tpu as pltpu
from jax.experimental.pallas import tpu_sc as plsc

BATCH = 4096
SEQ = 200
SEQ_PAD = 256
EMBED_DIM = 64
PAD_DIM = 128
VOCAB = 100000
ROWS = BATCH * SEQ
HALF = ROWS // 2
BHALF = BATCH // 2
NUM_CORES = 2
NUM_SUBCORES = 16
NW = NUM_CORES * NUM_SUBCORES
PPW = BHALF // NW
NPAIR = PPW // 2
GSUBS = ((0, 104), (104, 96))
LANES = 16
CPR = EMBED_DIM // LANES
TRI = 512
RB = 512


def _sc_body(tab, idx, pos, L2, pos_v,
             idxa0_v, idxb0_v, idxa1_v, idxb1_v,
             rowsa0_v, rowsb0_v, rowsa1_v, rowsb1_v,
             stg0_v, stg1_v,
             gsem0, gsem1, osem0, osem1, isem0, isem1):
    wid = lax.axis_index("s") * NUM_CORES + lax.axis_index("c")
    wbase = wid * PPW
    idxa_vs = (idxa0_v, idxa1_v)
    idxb_vs = (idxb0_v, idxb1_v)
    rowsa_vs = (rowsa0_v, rowsa1_v)
    rowsb_vs = (rowsb0_v, rowsb1_v)
    stg_vs = (stg0_v, stg1_v)
    gsems = (gsem0, gsem1)
    osems = (osem0, osem1)
    isems = (isem0, isem1)

    pltpu.sync_copy(pos, pos_v)

    def start_idx(c, b):
        bb = wbase + c
        pltpu.async_copy(
            idx.at[pl.ds(bb * SEQ_PAD, SEQ_PAD)], idxa_vs[b], isems[b])
        pltpu.async_copy(
            idx.at[pl.ds((BHALF + bb) * SEQ_PAD, SEQ_PAD)], idxb_vs[b],
            isems[b])

    def wait_idx(c, b):
        bb = wbase + c
        pltpu.make_async_copy(
            idx.at[pl.ds(bb * SEQ_PAD, SEQ_PAD)], idxa_vs[b],
            isems[b]).wait()
        pltpu.make_async_copy(
            idx.at[pl.ds((BHALF + bb) * SEQ_PAD, SEQ_PAD)], idxb_vs[b],
            isems[b]).wait()

    def start_gathers(b):
        for off, n in GSUBS:
            pltpu.async_copy(
                tab.at[idxa_vs[b].at[pl.ds(off, n)]],
                rowsa_vs[b].at[pl.ds(off, n)], gsems[b])
            pltpu.async_copy(
                tab.at[idxb_vs[b].at[pl.ds(off, n)]],
                rowsb_vs[b].at[pl.ds(off, n)], gsems[b])

    def wait_gathers(b):
        pltpu.make_async_copy(
            tab.at[pl.ds(0, SEQ)], rowsa_vs[b], gsems[b]).wait()
        pltpu.make_async_copy(
            tab.at[pl.ds(0, SEQ)], rowsb_vs[b], gsems[b]).wait()

    def start_out(c, b):
        pltpu.async_copy(
            stg_vs[b], L2.at[pl.ds((wbase + c) * SEQ, SEQ)], osems[b])

    def wait_out(c, b):
        pltpu.make_async_copy(
            stg_vs[b], L2.at[pl.ds((wbase + c) * SEQ, SEQ)],
            osems[b]).wait()

    def add_pos(b):
        rowsa_v = rowsa_vs[b]
        rowsb_v = rowsb_vs[b]
        stg_v = stg_vs[b]

        def row_body(r, _):
            for cc in range(CPR):
                sl = pl.ds(cc * LANES, LANES)
                p = pos_v[r, sl]
                stg_v[r, sl] = rowsa_v[r, sl] + p
                stg_v[r, pl.ds(EMBED_DIM + cc * LANES, LANES)] = (
                    rowsb_v[r, sl] + p)
            return 0

        lax.fori_loop(0, SEQ, row_body, 0)

    for b in range(2):
        start_idx(b, b)
    for b in range(2):
        wait_idx(b, b)
        start_gathers(b)

    def pair_body(k, _):
        for b in range(2):
            c = 2 * k + b
            wait_gathers(b)

            @pl.when(k < NPAIR - 1)
            def _(c=c, b=b):
                start_idx(c + 2, b)

            @pl.when(k > 0)
            def _(c=c, b=b):
                wait_out(c - 2, b)

            add_pos(b)
            start_out(c, b)

            @pl.when(k < NPAIR - 1)
            def _(c=c, b=b):
                wait_idx(c + 2, b)
                start_gathers(b)

        return 0

    lax.fori_loop(0, NPAIR, pair_body, 0)
    wait_out(PPW - 2, 0)
    wait_out(PPW - 1, 1)


def _idxpad_body(i_ref, o_ref):
    o_ref[:, :SEQ] = i_ref[...]


def _depad_body(l_ref, o_ref):
    j = pl.program_id(1)

    @pl.when(j == 0)
    def _():
        o_ref[...] = l_ref[:, :EMBED_DIM]

    @pl.when(j == 1)
    def _():
        o_ref[...] = l_ref[:, EMBED_DIM:]


def _branch(tab, idx_flat, pos):
    mesh = plsc.VectorSubcoreMesh(core_axis_name="c", subcore_axis_name="s")
    gather = functools.partial(
        pl.kernel,
        mesh=mesh,
        compiler_params=pltpu.CompilerParams(use_tc_tiling_on_sc=False),
        out_type=jax.ShapeDtypeStruct((HALF, PAD_DIM), jnp.float32),
        scratch_types=[
            pltpu.VMEM((SEQ, EMBED_DIM), jnp.float32),
        ] + [pltpu.VMEM((SEQ_PAD,), jnp.int32)] * 4
          + [pltpu.VMEM((SEQ, EMBED_DIM), jnp.float32)] * 4
          + [pltpu.VMEM((SEQ, PAD_DIM), jnp.float32)] * 2
          + [pltpu.SemaphoreType.DMA] * 6,
    )(_sc_body)
    L2 = gather(tab, idx_flat, pos)

    out = pl.pallas_call(
        _depad_body,
        grid=(HALF // RB, 2),
        in_specs=[pl.BlockSpec((RB, PAD_DIM), lambda i, j: (i, 0))],
        out_specs=pl.BlockSpec(
            (RB, EMBED_DIM), lambda i, j: (j * (HALF // RB) + i, 0)),
        out_shape=jax.ShapeDtypeStruct((ROWS, EMBED_DIM), jnp.float32),
    )(L2)
    return out.reshape(BATCH, SEQ, EMBED_DIM)


def _pad_idx(tokens):
    idx256 = pl.pallas_call(
        _idxpad_body,
        grid=(BATCH // TRI,),
        in_specs=[pl.BlockSpec((TRI, SEQ), lambda i: (i, 0))],
        out_specs=pl.BlockSpec((TRI, SEQ_PAD), lambda i: (i, 0)),
        out_shape=jax.ShapeDtypeStruct((BATCH, SEQ_PAD), jnp.int32),
    )(tokens.astype(jnp.int32))
    return idx256.reshape(BATCH * SEQ_PAD)


@jax.jit
def kernel(g_tok_table, e_tok_table, g_pos_table, e_pos_table,
           g_text_tokens, e_text_tokens):
    g_idx = _pad_idx(g_text_tokens)
    e_idx = _pad_idx(e_text_tokens)
    g_out = _branch(g_tok_table, g_idx, g_pos_table)
    e_out = _branch(e_tok_table, e_idx, e_pos_table)
    return (g_out, e_out)


# speedup vs baseline: 1.0388x; 1.0388x over previous
import functools

import jax
import jax.numpy as jnp
from jax import lax
from jax.experimental import pallas as pl
from jax.experimental.pallas import tpu as pltpu
from jax.experimental.pallas import tpu_sc as plsc

BATCH = 4096
SEQ = 200
SEQ_PAD = 256
EMBED_DIM = 64
PAD_DIM = 128
VOCAB = 100000
ROWS = BATCH * SEQ
HALF = ROWS // 2
BHALF = BATCH // 2
NUM_CORES = 2
NUM_SUBCORES = 16
NW = NUM_CORES * NUM_SUBCORES
PPW = BHALF // NW
NTURN = PPW // 2
GSUBS = ((0, 104), (104, 96))
LANES = 16
CPR = EMBED_DIM // LANES
TRT = 1000
TRI = 512
RB = 512


def _sc_body(tab, idx, pos, L2, pos_v,
             idxa0_v, idxb0_v, idxa1_v, idxb1_v,
             rowsa_v, rowsb_v, stg0_v, stg1_v,
             gsem, osem0, osem1, isem):
    wid = lax.axis_index("s") * NUM_CORES + lax.axis_index("c")
    wbase = wid * PPW
    idxa_vs = (idxa0_v, idxa1_v)
    idxb_vs = (idxb0_v, idxb1_v)
    stg_vs = (stg0_v, stg1_v)
    osems = (osem0, osem1)

    pltpu.sync_copy(pos, pos_v)

    def start_idx(c, p):
        bb = wbase + c
        pltpu.async_copy(
            idx.at[pl.ds(bb * SEQ_PAD, SEQ_PAD)], idxa_vs[p], isem)
        pltpu.async_copy(
            idx.at[pl.ds((BHALF + bb) * SEQ_PAD, SEQ_PAD)], idxb_vs[p], isem)

    def wait_idx(c, p):
        bb = wbase + c
        pltpu.make_async_copy(
            idx.at[pl.ds(bb * SEQ_PAD, SEQ_PAD)], idxa_vs[p], isem).wait()
        pltpu.make_async_copy(
            idx.at[pl.ds((BHALF + bb) * SEQ_PAD, SEQ_PAD)], idxb_vs[p],
            isem).wait()

    def start_gathers(p):
        for off, n in GSUBS:
            pltpu.async_copy(
                tab.at[idxa_vs[p].at[pl.ds(off, n)]],
                rowsa_v.at[pl.ds(off, n)], gsem)
            pltpu.async_copy(
                tab.at[idxb_vs[p].at[pl.ds(off, n)]],
                rowsb_v.at[pl.ds(off, n)], gsem)

    def wait_gathers():
        pltpu.make_async_copy(tab.at[pl.ds(0, SEQ)], rowsa_v, gsem).wait()
        pltpu.make_async_copy(tab.at[pl.ds(0, SEQ)], rowsb_v, gsem).wait()

    def start_out(c, b):
        pltpu.async_copy(
            stg_vs[b], L2.at[pl.ds((wbase + c) * SEQ, SEQ)], osems[b])

    def wait_out(c, b):
        pltpu.make_async_copy(
            stg_vs[b], L2.at[pl.ds((wbase + c) * SEQ, SEQ)],
            osems[b]).wait()

    def add_pos(b):
        stg_v = stg_vs[b]

        def row_body(r, _):
            for cc in range(CPR):
                sl = pl.ds(cc * LANES, LANES)
                p = pos_v[r, sl]
                stg_v[r, sl] = rowsa_v[r, sl] + p
                stg_v[r, pl.ds(EMBED_DIM + cc * LANES, LANES)] = (
                    rowsb_v[r, sl] + p)
            return 0

        lax.fori_loop(0, SEQ, row_body, 0)

    start_idx(0, 0)
    wait_idx(0, 0)
    start_gathers(0)

    def turn_body(k, _):
        for b in range(2):
            c = 2 * k + b
            p = b
            wait_gathers()

            @pl.when(c < PPW - 1)
            def _(c=c, p=p):
                start_idx(c + 1, 1 - p)

            @pl.when(c >= 2)
            def _(c=c, b=b):
                wait_out(c - 2, b)

            add_pos(b)
            start_out(c, b)

            @pl.when(c < PPW - 1)
            def _(c=c, p=p):
                wait_idx(c + 1, 1 - p)
                start_gathers(1 - p)

        return 0

    lax.fori_loop(0, NTURN, turn_body, 0)
    wait_out(PPW - 2, 0)
    wait_out(PPW - 1, 1)


def _tabpad_body(t_ref, o_ref):
    o_ref[:, :EMBED_DIM] = t_ref[...]


def _idxpad_body(i_ref, o_ref):
    o_ref[:, :SEQ] = i_ref[...]


def _depad_body(l_ref, o_ref):
    j = pl.program_id(1)

    @pl.when(j == 0)
    def _():
        o_ref[...] = l_ref[:, :EMBED_DIM]

    @pl.when(j == 1)
    def _():
        o_ref[...] = l_ref[:, EMBED_DIM:]


def _branch(tab, idx_flat, pos):
    tab2 = pl.pallas_call(
        _tabpad_body,
        grid=(VOCAB // TRT,),
        in_specs=[pl.BlockSpec((TRT, EMBED_DIM), lambda i: (i, 0))],
        out_specs=pl.BlockSpec((TRT, PAD_DIM), lambda i: (i, 0)),
        out_shape=jax.ShapeDtypeStruct((VOCAB, PAD_DIM), jnp.float32),
    )(tab)

    mesh = plsc.VectorSubcoreMesh(core_axis_name="c", subcore_axis_name="s")
    gather = functools.partial(
        pl.kernel,
        mesh=mesh,
        out_type=jax.ShapeDtypeStruct((HALF, PAD_DIM), jnp.float32),
        scratch_types=[
            pltpu.VMEM((SEQ, EMBED_DIM), jnp.float32),
        ] + [pltpu.VMEM((SEQ_PAD,), jnp.int32)] * 4
          + [pltpu.VMEM((SEQ, PAD_DIM), jnp.float32)] * 4
          + [pltpu.SemaphoreType.DMA] * 4,
    )(_sc_body)
    L2 = gather(tab2, idx_flat, pos)

    out = pl.pallas_call(
        _depad_body,
        grid=(HALF // RB, 2),
        in_specs=[pl.BlockSpec((RB, PAD_DIM), lambda i, j: (i, 0))],
        out_specs=pl.BlockSpec(
            (RB, EMBED_DIM), lambda i, j: (j * (HALF // RB) + i, 0)),
        out_shape=jax.ShapeDtypeStruct((ROWS, EMBED_DIM), jnp.float32),
    )(L2)
    return out.reshape(BATCH, SEQ, EMBED_DIM)


def _pad_idx(tokens):
    idx256 = pl.pallas_call(
        _idxpad_body,
        grid=(BATCH // TRI,),
        in_specs=[pl.BlockSpec((TRI, SEQ), lambda i: (i, 0))],
        out_specs=pl.BlockSpec((TRI, SEQ_PAD), lambda i: (i, 0)),
        out_shape=jax.ShapeDtypeStruct((BATCH, SEQ_PAD), jnp.int32),
    )(tokens.astype(jnp.int32))
    return idx256.reshape(BATCH * SEQ_PAD)


@jax.jit
def kernel(g_tok_table, e_tok_table, g_pos_table, e_pos_table,
           g_text_tokens, e_text_tokens):
    g_idx = _pad_idx(g_text_tokens)
    e_idx = _pad_idx(e_text_tokens)
    g_out = _branch(g_tok_table, g_idx, g_pos_table)
    e_out = _branch(e_tok_table, e_idx, e_pos_table)
    return (g_out, e_out)


# speedup vs baseline: 1.0433x; 1.0044x over previous
import functools

import jax
import jax.numpy as jnp
from jax import lax
from jax.experimental import pallas as pl
from jax.experimental.pallas import tpu as pltpu
from jax.experimental.pallas import tpu_sc as plsc

BATCH = 4096
SEQ = 200
SEQ_PAD = 256
EMBED_DIM = 64
PAD_DIM = 128
VOCAB = 100000
ROWS = BATCH * SEQ
HALF = ROWS // 2
BHALF = BATCH // 2
NUM_CORES = 2
NUM_SUBCORES = 16
NW = NUM_CORES * NUM_SUBCORES
PPW = BHALF // NW
NTURN = PPW // 2
GSUBS = ((0, 104), (104, 96))
LANES = 16
CPR = EMBED_DIM // LANES
TRT = 1000
TRI = 512
RB = 512


def _sc_body(tab, idx1, idx2, pos, L2, pos_v,
             idxa0_v, idxb0_v, idxa1_v, idxb1_v,
             rowsa_v, rowsb_v, stg0_v, stg1_v,
             gsem, osem0, osem1, isem):
    wid = lax.axis_index("s") * NUM_CORES + lax.axis_index("c")
    wbase = wid * PPW
    idxa_vs = (idxa0_v, idxa1_v)
    idxb_vs = (idxb0_v, idxb1_v)
    stg_vs = (stg0_v, stg1_v)
    osems = (osem0, osem1)

    pltpu.sync_copy(pos, pos_v)

    def idx_copies(c, p):
        bb = wbase + c
        yield idx1.at[pl.ds(bb * 128, 128)], idxa_vs[p].at[pl.ds(0, 128)]
        yield idx2.at[pl.ds(bb * 128, 80)], idxa_vs[p].at[pl.ds(128, 80)]
        bb = BHALF + bb
        yield idx1.at[pl.ds(bb * 128, 128)], idxb_vs[p].at[pl.ds(0, 128)]
        yield idx2.at[pl.ds(bb * 128, 80)], idxb_vs[p].at[pl.ds(128, 80)]

    def start_idx(c, p):
        for src, dst in idx_copies(c, p):
            pltpu.async_copy(src, dst, isem)

    def wait_idx(c, p):
        for src, dst in idx_copies(c, p):
            pltpu.make_async_copy(src, dst, isem).wait()

    def start_gathers(p):
        for idx_v, rows_v in ((idxa_vs[p], rowsa_v), (idxb_vs[p], rowsb_v)):
            pltpu.async_copy(
                tab.at[idx_v.at[pl.ds(0, 128)]],
                rows_v.at[pl.ds(0, 128)], gsem)
            pltpu.async_copy(
                tab.at[idx_v.at[pl.ds(128, SEQ - 128)]],
                rows_v.at[pl.ds(128, SEQ - 128)], gsem)

    def wait_gathers():
        pltpu.make_async_copy(tab.at[pl.ds(0, SEQ)], rowsa_v, gsem).wait()
        pltpu.make_async_copy(tab.at[pl.ds(0, SEQ)], rowsb_v, gsem).wait()

    def start_out(c, b):
        pltpu.async_copy(
            stg_vs[b], L2.at[pl.ds((wbase + c) * SEQ, SEQ)], osems[b])

    def wait_out(c, b):
        pltpu.make_async_copy(
            stg_vs[b], L2.at[pl.ds((wbase + c) * SEQ, SEQ)],
            osems[b]).wait()

    def add_pos(b):
        stg_v = stg_vs[b]

        def row_body(r, _):
            for cc in range(CPR):
                sl = pl.ds(cc * LANES, LANES)
                p = pos_v[r, sl]
                stg_v[r, sl] = rowsa_v[r, sl] + p
                stg_v[r, pl.ds(EMBED_DIM + cc * LANES, LANES)] = (
                    rowsb_v[r, sl] + p)
            return 0

        lax.fori_loop(0, SEQ, row_body, 0)

    start_idx(0, 0)
    wait_idx(0, 0)
    start_gathers(0)

    def turn_body(k, _):
        for b in range(2):
            c = 2 * k + b
            p = b
            wait_gathers()

            @pl.when(c < PPW - 1)
            def _(c=c, p=p):
                start_idx(c + 1, 1 - p)

            @pl.when(c >= 2)
            def _(c=c, b=b):
                wait_out(c - 2, b)

            add_pos(b)
            start_out(c, b)

            @pl.when(c < PPW - 1)
            def _(c=c, p=p):
                wait_idx(c + 1, 1 - p)
                start_gathers(1 - p)

        return 0

    lax.fori_loop(0, NTURN, turn_body, 0)
    wait_out(PPW - 2, 0)
    wait_out(PPW - 1, 1)


def _tabpad_body(t_ref, o_ref):
    o_ref[:, :EMBED_DIM] = t_ref[...]


def _idxpad_body(i_ref, o1_ref, o2_ref):
    o1_ref[...] = i_ref[:, :128]
    o2_ref[:, :SEQ - 128] = i_ref[:, 128:]


def _depad_body(l_ref, o_ref):
    j = pl.program_id(1)

    @pl.when(j == 0)
    def _():
        o_ref[...] = l_ref[:, :EMBED_DIM]

    @pl.when(j == 1)
    def _():
        o_ref[...] = l_ref[:, EMBED_DIM:]


def _branch(tab, idx_flat, pos):
    tab2 = pl.pallas_call(
        _tabpad_body,
        grid=(VOCAB // TRT,),
        in_specs=[pl.BlockSpec((TRT, EMBED_DIM), lambda i: (i, 0))],
        out_specs=pl.BlockSpec((TRT, PAD_DIM), lambda i: (i, 0)),
        out_shape=jax.ShapeDtypeStruct((VOCAB, PAD_DIM), jnp.float32),
    )(tab)

    mesh = plsc.VectorSubcoreMesh(core_axis_name="c", subcore_axis_name="s")
    gather = functools.partial(
        pl.kernel,
        mesh=mesh,
        out_type=jax.ShapeDtypeStruct((HALF, PAD_DIM), jnp.float32),
        scratch_types=[
            pltpu.VMEM((SEQ, EMBED_DIM), jnp.float32),
        ] + [pltpu.VMEM((SEQ_PAD,), jnp.int32)] * 4
          + [pltpu.VMEM((SEQ, PAD_DIM), jnp.float32)] * 4
          + [pltpu.SemaphoreType.DMA] * 4,
    )(_sc_body)
    L2 = gather(tab2, idx_flat[0], idx_flat[1], pos)

    out = pl.pallas_call(
        _depad_body,
        grid=(HALF // RB, 2),
        in_specs=[pl.BlockSpec((RB, PAD_DIM), lambda i, j: (i, 0))],
        out_specs=pl.BlockSpec(
            (RB, EMBED_DIM), lambda i, j: (j * (HALF // RB) + i, 0)),
        out_shape=jax.ShapeDtypeStruct((ROWS, EMBED_DIM), jnp.float32),
    )(L2)
    return out.reshape(BATCH, SEQ, EMBED_DIM)


def _pad_idx(tokens):
    idx1, idx2 = pl.pallas_call(
        _idxpad_body,
        grid=(BATCH // TRI,),
        in_specs=[pl.BlockSpec((TRI, SEQ), lambda i: (i, 0))],
        out_specs=(pl.BlockSpec((TRI, 128), lambda i: (i, 0)),
                   pl.BlockSpec((TRI, 128), lambda i: (i, 0))),
        out_shape=(jax.ShapeDtypeStruct((BATCH, 128), jnp.int32),
                   jax.ShapeDtypeStruct((BATCH, 128), jnp.int32)),
    )(tokens.astype(jnp.int32))
    return idx1.reshape(BATCH * 128), idx2.reshape(BATCH * 128)


@jax.jit
def kernel(g_tok_table, e_tok_table, g_pos_table, e_pos_table,
           g_text_tokens, e_text_tokens):
    g_idx = _pad_idx(g_text_tokens)
    e_idx = _pad_idx(e_text_tokens)
    g_out = _branch(g_tok_table, g_idx, g_pos_table)
    e_out = _branch(e_tok_table, e_idx, e_pos_table)
    return (g_out, e_out)


# speedup vs baseline: 1.8119x; 1.7367x over previous
import functools

import jax
import jax.numpy as jnp
from jax import lax
from jax.experimental import pallas as pl
from jax.experimental.pallas import tpu as pltpu
from jax.experimental.pallas import tpu_sc as plsc

BATCH = 4096
SEQ = 200
SEQ_PAD = 256
EMBED_DIM = 64
PAD_DIM = 128
VOCAB = 100000
ROWS = BATCH * SEQ
HALF = ROWS // 2
BHALF = BATCH // 2
NUM_CORES = 2
NUM_SUBCORES = 16
NW = NUM_CORES * NUM_SUBCORES
PPW = BHALF // NW
NTURN = PPW // 2
GSUBS = ((0, 104), (104, 96))
LANES = 16
CPR = EMBED_DIM // LANES
TRT = 1000
TRI = 512
RB = 3200


def _sc_body(tab, idx1, idx2, pos, L2, pos_v,
             idxa0_v, idxb0_v, idxa1_v, idxb1_v,
             rowsa_v, rowsb_v, stg0_v, stg1_v,
             gsem, osem0, osem1, isem):
    wid = lax.axis_index("s") * NUM_CORES + lax.axis_index("c")
    wbase = wid * PPW
    idxa_vs = (idxa0_v, idxa1_v)
    idxb_vs = (idxb0_v, idxb1_v)
    stg_vs = (stg0_v, stg1_v)
    osems = (osem0, osem1)

    pltpu.sync_copy(pos, pos_v)

    def idx_copies(c, p):
        bb = wbase + c
        yield idx1.at[pl.ds(bb * 128, 128)], idxa_vs[p].at[pl.ds(0, 128)]
        yield idx2.at[pl.ds(bb * 128, 80)], idxa_vs[p].at[pl.ds(128, 80)]
        bb = BHALF + bb
        yield idx1.at[pl.ds(bb * 128, 128)], idxb_vs[p].at[pl.ds(0, 128)]
        yield idx2.at[pl.ds(bb * 128, 80)], idxb_vs[p].at[pl.ds(128, 80)]

    def start_idx(c, p):
        for src, dst in idx_copies(c, p):
            pltpu.async_copy(src, dst, isem)

    def wait_idx(c, p):
        for src, dst in idx_copies(c, p):
            pltpu.make_async_copy(src, dst, isem).wait()

    def start_gathers(p):
        for idx_v, rows_v in ((idxa_vs[p], rowsa_v), (idxb_vs[p], rowsb_v)):
            pltpu.async_copy(
                tab.at[idx_v.at[pl.ds(0, 128)]],
                rows_v.at[pl.ds(0, 128)], gsem)
            pltpu.async_copy(
                tab.at[idx_v.at[pl.ds(128, SEQ - 128)]],
                rows_v.at[pl.ds(128, SEQ - 128)], gsem)

    def wait_gathers():
        pltpu.make_async_copy(tab.at[pl.ds(0, SEQ)], rowsa_v, gsem).wait()
        pltpu.make_async_copy(tab.at[pl.ds(0, SEQ)], rowsb_v, gsem).wait()

    def start_out(c, b):
        pltpu.async_copy(
            stg_vs[b], L2.at[pl.ds((wbase + c) * SEQ, SEQ)], osems[b])

    def wait_out(c, b):
        pltpu.make_async_copy(
            stg_vs[b], L2.at[pl.ds((wbase + c) * SEQ, SEQ)],
            osems[b]).wait()

    def add_pos(b):
        stg_v = stg_vs[b]

        def row_body(r, _):
            for cc in range(CPR):
                sl = pl.ds(cc * LANES, LANES)
                p = pos_v[r, sl]
                stg_v[r, sl] = rowsa_v[r, sl] + p
                stg_v[r, pl.ds(EMBED_DIM + cc * LANES, LANES)] = (
                    rowsb_v[r, sl] + p)
            return 0

        lax.fori_loop(0, SEQ, row_body, 0)

    start_idx(0, 0)
    wait_idx(0, 0)
    start_gathers(0)

    def turn_body(k, _):
        for b in range(2):
            c = 2 * k + b
            p = b
            wait_gathers()

            @pl.when(c < PPW - 1)
            def _(c=c, p=p):
                start_idx(c + 1, 1 - p)

            @pl.when(c >= 2)
            def _(c=c, b=b):
                wait_out(c - 2, b)

            add_pos(b)
            start_out(c, b)

            @pl.when(c < PPW - 1)
            def _(c=c, p=p):
                wait_idx(c + 1, 1 - p)
                start_gathers(1 - p)

        return 0

    lax.fori_loop(0, NTURN, turn_body, 0)
    wait_out(PPW - 2, 0)
    wait_out(PPW - 1, 1)


def _tabpad_body(t_ref, o_ref):
    o_ref[:, :EMBED_DIM] = t_ref[...]


def _idxpad_body(i_ref, o1_ref, o2_ref):
    o1_ref[...] = i_ref[:, :128]
    o2_ref[:, :SEQ - 128] = i_ref[:, 128:]


def _depad_body(l_ref, o_ref):
    j = pl.program_id(1)

    @pl.when(j == 0)
    def _():
        o_ref[...] = l_ref[:, :EMBED_DIM]

    @pl.when(j == 1)
    def _():
        o_ref[...] = l_ref[:, EMBED_DIM:]


def _branch(tab, idx_flat, pos):
    tab2 = pl.pallas_call(
        _tabpad_body,
        grid=(VOCAB // TRT,),
        in_specs=[pl.BlockSpec((TRT, EMBED_DIM), lambda i: (i, 0))],
        out_specs=pl.BlockSpec((TRT, PAD_DIM), lambda i: (i, 0)),
        out_shape=jax.ShapeDtypeStruct((VOCAB, PAD_DIM), jnp.float32),
    )(tab)

    mesh = plsc.VectorSubcoreMesh(core_axis_name="c", subcore_axis_name="s")
    gather = functools.partial(
        pl.kernel,
        mesh=mesh,
        out_type=jax.ShapeDtypeStruct((HALF, PAD_DIM), jnp.float32),
        scratch_types=[
            pltpu.VMEM((SEQ, EMBED_DIM), jnp.float32),
        ] + [pltpu.VMEM((SEQ_PAD,), jnp.int32)] * 4
          + [pltpu.VMEM((SEQ, PAD_DIM), jnp.float32)] * 4
          + [pltpu.SemaphoreType.DMA] * 4,
    )(_sc_body)
    L2 = gather(tab2, idx_flat[0], idx_flat[1], pos)

    out = pl.pallas_call(
        _depad_body,
        grid=(HALF // RB, 2),
        in_specs=[pl.BlockSpec((RB, PAD_DIM), lambda i, j: (i, 0))],
        out_specs=pl.BlockSpec(
            (RB, EMBED_DIM), lambda i, j: (j * (HALF // RB) + i, 0)),
        out_shape=jax.ShapeDtypeStruct((ROWS, EMBED_DIM), jnp.float32),
    )(L2)
    return out.reshape(BATCH, SEQ, EMBED_DIM)


def _pad_idx(tokens):
    idx1, idx2 = pl.pallas_call(
        _idxpad_body,
        grid=(BATCH // TRI,),
        in_specs=[pl.BlockSpec((TRI, SEQ), lambda i: (i, 0))],
        out_specs=(pl.BlockSpec((TRI, 128), lambda i: (i, 0)),
                   pl.BlockSpec((TRI, 128), lambda i: (i, 0))),
        out_shape=(jax.ShapeDtypeStruct((BATCH, 128), jnp.int32),
                   jax.ShapeDtypeStruct((BATCH, 128), jnp.int32)),
    )(tokens.astype(jnp.int32))
    return idx1.reshape(BATCH * 128), idx2.reshape(BATCH * 128)


@jax.jit
def kernel(g_tok_table, e_tok_table, g_pos_table, e_pos_table,
           g_text_tokens, e_text_tokens):
    g_idx = _pad_idx(g_text_tokens)
    e_idx = _pad_idx(e_text_tokens)
    g_out = _branch(g_tok_table, g_idx, g_pos_table)
    e_out = _branch(e_tok_table, e_idx, e_pos_table)
    return (g_out, e_out)
